# baseline (speedup 1.0000x reference)
import functools

import jax
import jax.numpy as jnp
from jax import lax
from jax.experimental import pallas as pl
from jax.experimental.pallas import tpu as pltpu
from jax.experimental.pallas import tpu_sc as plsc

NY, NX = 512, 512
NW = 32
CHUNK = 128
WIDE = 128


def _sc_scatter(pf_flat, idx_2d):
    n, _ = pf_flat.shape
    nb = n // 32768
    rows_total = nb * NY * NX
    per_w = n // NW
    n_sub = per_w // CHUNK
    p = n // nb
    pix_w = rows_total // NW
    rows_w = pix_w // NX
    sub_per_b = NW // nb

    mesh = plsc.VectorSubcoreMesh(core_axis_name="c", subcore_axis_name="s")

    @functools.partial(
        pl.kernel,
        mesh=mesh,
        out_type=(
            jax.ShapeDtypeStruct((nb, NY, NX), jnp.int32),
            jax.ShapeDtypeStruct((rows_total, WIDE), jnp.float32),
        ),
        scratch_types=[
            pltpu.VMEM((rows_w, NX), jnp.int32),
            pltpu.VMEM((32, CHUNK), jnp.int32),
            pltpu.VMEM((n_sub, CHUNK), jnp.int32),
            pltpu.VMEM((4 * CHUNK, WIDE), jnp.float32),
            pltpu.SemaphoreType.DMA,
        ],
        compiler_params=pltpu.CompilerParams(needs_layout_passes=False),
    )
    def scatter_kernel(pf_hbm, idx2_hbm, mask_hbm, out_hbm,
                       mask_v, bidx_v, sidx_v, rows_v, sem):
        wid = lax.axis_index("s") * 2 + lax.axis_index("c")
        batch = wid // sub_per_b
        pix_base = wid * pix_w

        zeros16 = jnp.zeros((16,), jnp.int32)
        ones16 = jnp.ones((16,), jnp.int32)

        def zero_body(i, carry):
            r = i // (NX // 16)
            j = i % (NX // 16)
            mask_v[r, pl.ds(j * 16, 16)] = zeros16
            return carry

        lax.fori_loop(0, pix_w // 16, zero_body, 0)

        rows_per_b = p // CHUNK

        def mask_stage(s, carry):
            soff = pl.multiple_of(batch * rows_per_b + s * 32, 32)
            pltpu.sync_copy(idx2_hbm.at[pl.ds(soff, 32)], bidx_v)

            def mask_body(i, carry2):
                v = bidx_v[i // (CHUNK // 16), pl.ds((i % (CHUNK // 16)) * 16, 16)]
                pos = v - pix_base
                m = (pos >= 0) & (pos < pix_w)
                plsc.store_scatter(
                    mask_v,
                    [lax.shift_right_logical(pos, 9), pos & (NX - 1)],
                    ones16,
                    mask=m,
                )
                return carry2

            lax.fori_loop(0, (32 * CHUNK) // 16, mask_body, 0)
            return carry

        lax.fori_loop(0, rows_per_b // 32, mask_stage, 0)
        pltpu.sync_copy(
            mask_v, mask_hbm.at[batch, pl.ds((wid % sub_per_b) * rows_w, rows_w)]
        )

        pltpu.sync_copy(idx2_hbm.at[pl.ds(wid * n_sub, n_sub)], sidx_v)

        def scat_body(j, carry):
            off = pl.multiple_of(wid * per_w + j * (4 * CHUNK), 4 * CHUNK)
            pltpu.sync_copy(pf_hbm.at[pl.ds(off, 4 * CHUNK)], rows_v)
            copies = [
                pltpu.async_copy(
                    rows_v.at[pl.ds(q * CHUNK, CHUNK)],
                    out_hbm.at[sidx_v.at[j * 4 + q]],
                    sem,
                )
                for q in range(4)
            ]
            for cp in copies:
                cp.wait()
            return carry

        lax.fori_loop(0, n_sub // 4, scat_body, 0)

    return scatter_kernel(pf_flat, idx_2d)


def _tc_transpose(mask_img, canvas_nhwc, c):
    b = canvas_nhwc.shape[0]
    rows = 64

    def body(mask_ref, in_ref, out_ref):
        m = mask_ref[0] != 0
        x = in_ref[0][:, :c]
        xt = x.reshape(rows, NX, c).transpose(2, 0, 1)
        out_ref[0] = jnp.where(m[None], xt, jnp.float32(0.0))

    return pl.pallas_call(
        body,
        grid=(b, NY // rows),
        in_specs=[
            pl.BlockSpec((1, rows, NX), lambda i, j: (i, j, 0)),
            pl.BlockSpec((1, rows * NX, WIDE), lambda i, j: (i, j, 0)),
        ],
        out_specs=pl.BlockSpec((1, c, rows, NX), lambda i, j: (i, 0, j, 0)),
        out_shape=jax.ShapeDtypeStruct((b, c, NY, NX), jnp.float32),
    )(mask_img, canvas_nhwc)


@jax.jit
def kernel(pillar_features, coords):
    b, p, c = pillar_features.shape
    y = coords[:, :, 2].astype(jnp.int32)
    x = coords[:, :, 3].astype(jnp.int32)
    idx_global = (
        jnp.arange(b, dtype=jnp.int32)[:, None] * (NY * NX) + y * NX + x
    ).reshape(-1)
    pf_flat = jnp.pad(
        pillar_features.reshape(b * p, c), ((0, 0), (0, WIDE - c))
    )
    mask, flat = _sc_scatter(pf_flat, idx_global.reshape(-1, CHUNK))
    return _tc_transpose(mask, flat.reshape(b, NY * NX, WIDE), c)
